# Initial kernel scaffold; baseline (speedup 1.0000x reference)
import jax
import jax.numpy as jnp
from jax.experimental import pallas as pl


def kernel(x, edge_index, batch, W1, b1, W2, b2, W3, b3, Wfc, bfc):
    raise NotImplementedError("write your pallas kernel here")


# speedup vs baseline: 3.8863x; 3.8863x over previous
import functools

import jax
import jax.numpy as jnp
from jax import lax
from jax.experimental import pallas as pl
from jax.experimental.pallas import tpu as pltpu
from jax.experimental.pallas import tpu_sc as plsc

N = 10000
E = 320000
D = 128
C = 10
G = 128

NC = 2
NS = 16
NW = NC * NS

CH = 128
NCHUNKS = 80
NPHASE = 2
CPP = NCHUNKS // NPHASE
E_PAD = NW * NCHUNKS * CH
N_PAD = 10240
ROWS_PER_TILE = N_PAD // NS
COPY_CHUNKS = ROWS_PER_TILE // CH
NB_POOL = N_PAD // 128


def _segsum_body(h_hbm, src_hbm, dst_hbm, out_hbm,
                 src_v, dst_v, buf0, buf1, agg_sh, sem0, sem1):
    c = lax.axis_index("c")
    s = lax.axis_index("s")
    wid = s * NC + c

    zero16 = jnp.zeros((16,), jnp.float32)

    def _zrow(r, carry):
        for k in range(D // 16):
            buf0[r, pl.ds(k * 16, 16)] = zero16
        return carry

    lax.fori_loop(0, CH, _zrow, 0)
    row0 = s * ROWS_PER_TILE
    for t in range(COPY_CHUNKS):
        pltpu.sync_copy(buf0, agg_sh.at[pl.ds(row0 + t * CH, CH)])
    plsc.subcore_barrier()

    bufs = (buf0, buf1)
    sems = (sem0, sem1)
    for p in range(NPHASE):
        pltpu.sync_copy(src_hbm.at[wid, pl.ds(p * CPP, CPP)], src_v)
        pltpu.sync_copy(dst_hbm.at[wid, pl.ds(p * CPP, CPP)], dst_v)
        descs = [None, None]
        descs[0] = pltpu.async_copy(h_hbm.at[src_v.at[0]], buf0, sem0)
        for j in range(CPP):
            b = j % 2
            descs[b].wait()
            nj = j + 1
            if nj < CPP:
                nb = nj % 2
                descs[nb] = pltpu.async_copy(
                    h_hbm.at[src_v.at[nj]], bufs[nb], sems[nb])
            pltpu.sync_copy(bufs[b], agg_sh.at[dst_v.at[j]], add=True)
    plsc.subcore_barrier()

    for t in range(COPY_CHUNKS):
        rr = row0 + t * CH
        pltpu.sync_copy(agg_sh.at[pl.ds(rr, CH)], buf0)
        pltpu.sync_copy(buf0, out_hbm.at[c, pl.ds(rr, CH)])


_segsum = functools.partial(
    pl.kernel,
    out_type=jax.ShapeDtypeStruct((NC, N_PAD, D), jnp.float32),
    mesh=plsc.VectorSubcoreMesh(core_axis_name="c", subcore_axis_name="s"),
    scratch_types=[
        pltpu.VMEM((CPP, CH), jnp.int32),
        pltpu.VMEM((CPP, CH), jnp.int32),
        pltpu.VMEM((CH, D), jnp.float32),
        pltpu.VMEM((CH, D), jnp.float32),
        pltpu.VMEM_SHARED((N_PAD, D), jnp.float32),
        pltpu.SemaphoreType.DMA,
        pltpu.SemaphoreType.DMA,
    ],
)(_segsum_body)


def _mm_body(parts_ref, w_ref, b_ref, o_ref):
    acc = parts_ref[0] + parts_ref[1]
    y = jnp.dot(acc, w_ref[...], preferred_element_type=jnp.float32)
    o_ref[...] = jnp.maximum(y + b_ref[...], 0.0)


def _layer_mm(parts, W, b):
    blk = 1024
    return pl.pallas_call(
        _mm_body,
        grid=(N_PAD // blk,),
        in_specs=[
            pl.BlockSpec((NC, blk, D), lambda i: (0, i, 0)),
            pl.BlockSpec((D, D), lambda i: (0, 0)),
            pl.BlockSpec((1, D), lambda i: (0, 0)),
        ],
        out_specs=pl.BlockSpec((blk, D), lambda i: (i, 0)),
        out_shape=jax.ShapeDtypeStruct((N_PAD, D), jnp.float32),
    )(parts, W, b.reshape(1, D))


def _pool_body(batch_ref, h_ref, wfc_ref, bfc_ref, o_ref, sums_ref, cnt_ref):
    i = pl.program_id(0)

    @pl.when(i == 0)
    def _():
        sums_ref[...] = jnp.zeros_like(sums_ref)
        cnt_ref[...] = jnp.zeros_like(cnt_ref)

    bvec = batch_ref[0]
    gid = lax.broadcasted_iota(jnp.int32, (G, 128), 0)
    mask = (gid == bvec).astype(jnp.float32)
    sums_ref[...] += jnp.dot(mask, h_ref[...], preferred_element_type=jnp.float32)
    cnt_ref[...] += jnp.sum(mask, axis=1, keepdims=True)

    @pl.when(i == NB_POOL - 1)
    def _():
        pooled = sums_ref[...] / jnp.maximum(cnt_ref[...], 1.0)
        logits = jnp.dot(pooled, wfc_ref[...], preferred_element_type=jnp.float32)
        logits = logits + bfc_ref[...]
        col = lax.broadcasted_iota(jnp.int32, (G, D), 1)
        valid = col < C
        neg = jnp.where(valid, logits, -jnp.inf)
        m = jnp.max(neg, axis=1, keepdims=True)
        ex = jnp.where(valid, jnp.exp(logits - m), 0.0)
        lse = jnp.log(jnp.sum(ex, axis=1, keepdims=True)) + m
        o_ref[...] = logits - lse


def _pool_head(batch3, h, wfc_p, bfc_p):
    return pl.pallas_call(
        _pool_body,
        grid=(NB_POOL,),
        in_specs=[
            pl.BlockSpec((1, 1, 128), lambda i: (i, 0, 0)),
            pl.BlockSpec((128, D), lambda i: (i, 0)),
            pl.BlockSpec((D, D), lambda i: (0, 0)),
            pl.BlockSpec((1, D), lambda i: (0, 0)),
        ],
        out_specs=pl.BlockSpec((G, D), lambda i: (0, 0)),
        out_shape=jax.ShapeDtypeStruct((G, D), jnp.float32),
        scratch_shapes=[
            pltpu.VMEM((G, D), jnp.float32),
            pltpu.VMEM((G, D), jnp.float32),
        ],
    )(batch3, h, wfc_p, bfc_p)


def kernel(x, edge_index, batch, W1, b1, W2, b2, W3, b3, Wfc, bfc):
    src = edge_index[0]
    dst = edge_index[1]
    pad = E_PAD - E
    pad_dst = N + jnp.arange(pad, dtype=jnp.int32) % (N_PAD - N)
    src_w = jnp.concatenate([src, jnp.zeros((pad,), jnp.int32)]).reshape(
        NW, NCHUNKS, CH)
    dst_w = jnp.concatenate([dst, pad_dst]).reshape(NW, NCHUNKS, CH)
    h = jnp.pad(x, ((0, N_PAD - N), (0, 0)))
    batch3 = jnp.pad(batch, (0, N_PAD - N), constant_values=G).reshape(
        NB_POOL, 1, 128)
    wfc_p = jnp.pad(Wfc, ((0, 0), (0, D - C)))
    bfc_p = jnp.pad(bfc, (0, D - C)).reshape(1, D)

    for (W, b) in ((W1, b1), (W2, b2), (W3, b3)):
        parts = _segsum(h, src_w, dst_w)
        h = _layer_mm(parts, W, b)
    out = _pool_head(batch3, h, wfc_p, bfc_p)
    return out[:, :C]
